# Initial kernel scaffold; baseline (speedup 1.0000x reference)
import functools

import jax
import jax.numpy as jnp
import numpy as np
from jax import lax
from jax.experimental import pallas as pl
from jax.experimental.pallas import tpu as pltpu
from jax.experimental.pallas import tpu_sc as plsc

NC = 2
NS = 16
NW = NC * NS
LANES = 16

CHUNK_ROWS = 32


def _make_sc_kernel(B, L, V, E):
    assert E == LANES
    rows_per_w = B // NW
    nchunks = rows_per_w // CHUNK_ROWS
    ci = CHUNK_ROWS * L
    idx_per_w = rows_per_w * L
    inv_l = 1.0 / float(L)

    mesh = plsc.VectorSubcoreMesh(core_axis_name="c", subcore_axis_name="s")

    @functools.partial(
        pl.kernel,
        mesh=mesh,
        out_type=jax.ShapeDtypeStruct((B,), jnp.float32),
        scratch_types=[
            pltpu.VMEM((ci,), jnp.int32),
            pltpu.VMEM((ci, E), jnp.float32),
            pltpu.VMEM((CHUNK_ROWS, E), jnp.float32),
            pltpu.VMEM((rows_per_w,), jnp.float32),
            pltpu.VMEM((E,), jnp.float32),
            pltpu.VMEM((E,), jnp.float32),
            pltpu.VMEM((ci,), jnp.int32),
            pltpu.SemaphoreType.DMA,
        ],
    )
    def sc_kernel(x_hbm, emb_hbm, w_hbm, b_hbm, oidx_hbm, out_hbm,
                  idx_v, vals_v, acc_v, outbuf, wv, bv, oidx_v, sem):
        wid = lax.axis_index("s") * NC + lax.axis_index("c")
        pltpu.sync_copy(w_hbm, wv)
        pltpu.sync_copy(b_hbm, bv)
        pltpu.sync_copy(oidx_hbm, oidx_v)
        base_i = wid * idx_per_w

        iota = lax.broadcasted_iota(jnp.int32, (LANES,), 0)
        zero = jnp.zeros((LANES,), jnp.float32)

        def chunk_body(c, carry):
            off = base_i + c * ci
            pltpu.sync_copy(x_hbm.at[pl.ds(off, ci)], idx_v)
            pltpu.async_copy(emb_hbm.at[idx_v], vals_v, sem).wait()
            for r in range(CHUNK_ROWS):
                acc_v[r] = zero
            pltpu.sync_copy(vals_v, acc_v.at[oidx_v], add=True)
            for g in range(CHUNK_ROWS // LANES):
                rows = g * LANES + iota
                z = jnp.zeros((LANES,), jnp.float32)
                for k in range(E):
                    col = plsc.load_gather(
                        acc_v, [rows, jnp.full((LANES,), k, jnp.int32)])
                    z = z + col * wv[k]
                z = z * inv_l + bv
                e = jnp.exp(-jnp.abs(z))
                s = jnp.where(z >= 0, 1.0 / (1.0 + e), e / (1.0 + e))
                outbuf[pl.ds(c * CHUNK_ROWS + g * LANES, LANES)] = s
            return carry

        lax.fori_loop(0, nchunks, chunk_body, 0)
        pltpu.sync_copy(outbuf, out_hbm.at[pl.ds(wid * rows_per_w, rows_per_w)])

    return sc_kernel


@jax.jit
def kernel(x, emb_table, lin_w, lin_b):
    B, L = x.shape
    V, E = emb_table.shape
    x_flat = x.reshape(-1).astype(jnp.int32)
    w16 = lin_w.reshape(E).astype(jnp.float32)
    b16 = jnp.broadcast_to(lin_b.reshape(1), (E,)).astype(jnp.float32)
    oidx = jnp.asarray(np.repeat(np.arange(CHUNK_ROWS), L), dtype=jnp.int32)
    out = _make_sc_kernel(B, L, V, E)(x_flat, emb_table, w16, b16, oidx)
    return out.reshape(B, 1)


# speedup vs baseline: 8.2631x; 8.2631x over previous
import functools

import jax
import jax.numpy as jnp
from jax import lax
from jax.experimental import pallas as pl
from jax.experimental.pallas import tpu as pltpu
from jax.experimental.pallas import tpu_sc as plsc

NC = 2
NS = 16
NW = NC * NS
LANES = 16

CHUNK_ROWS = 32


def _make_sc_kernel(B, L, V, E):
    assert E == LANES
    rows_per_w = B // NW
    nchunks = rows_per_w // CHUNK_ROWS
    ci = CHUNK_ROWS * L
    idx_per_w = rows_per_w * L
    inv_l = 1.0 / float(L)

    mesh = plsc.VectorSubcoreMesh(core_axis_name="c", subcore_axis_name="s")

    @functools.partial(
        pl.kernel,
        mesh=mesh,
        out_type=jax.ShapeDtypeStruct((B,), jnp.float32),
        compiler_params=pltpu.CompilerParams(
            needs_layout_passes=False, use_tc_tiling_on_sc=False),
        scratch_types=[
            pltpu.VMEM((ci,), jnp.int32),
            pltpu.VMEM((ci, E), jnp.float32),
            pltpu.VMEM((rows_per_w,), jnp.float32),
            pltpu.VMEM((E,), jnp.float32),
            pltpu.VMEM((E,), jnp.float32),
            pltpu.SemaphoreType.DMA,
        ],
    )
    def sc_kernel(x_hbm, emb_hbm, w_hbm, b_hbm, out_hbm,
                  idx_v, vals_v, outbuf, wv, bv, sem):
        wid = lax.axis_index("s") * NC + lax.axis_index("c")
        pltpu.sync_copy(w_hbm, wv)
        pltpu.sync_copy(b_hbm, bv)
        base_i = wid * idx_per_w

        iota = lax.broadcasted_iota(jnp.int32, (LANES,), 0)
        wvec = wv[...]
        bvec = bv[...]

        def chunk_body(c, carry):
            off = base_i + c * ci
            pltpu.sync_copy(x_hbm.at[pl.ds(off, ci)], idx_v)
            pltpu.async_copy(emb_hbm.at[idx_v], vals_v, sem).wait()

            def row_body(r, z):
                rb = r * L
                a0 = vals_v[rb]
                a1 = vals_v[rb + 1]
                a2 = vals_v[rb + 2]
                a3 = vals_v[rb + 3]
                for l in range(4, L, 4):
                    a0 = a0 + vals_v[rb + l]
                    a1 = a1 + vals_v[rb + l + 1]
                    a2 = a2 + vals_v[rb + l + 2]
                    a3 = a3 + vals_v[rb + l + 3]
                acc = (a0 + a1) + (a2 + a3)
                sj = jnp.sum(acc * wvec)
                return z + jnp.where(iota == (r % LANES), sj, 0.0)

            for g in range(CHUNK_ROWS // LANES):
                z = lax.fori_loop(g * LANES, (g + 1) * LANES, row_body,
                                  jnp.zeros((LANES,), jnp.float32))
                z = z * inv_l + bvec
                e = jnp.exp(-jnp.abs(z))
                s = jnp.where(z >= 0, 1.0 / (1.0 + e), e / (1.0 + e))
                outbuf[pl.ds(c * CHUNK_ROWS + g * LANES, LANES)] = s
            return carry

        lax.fori_loop(0, nchunks, chunk_body, 0)
        pltpu.sync_copy(outbuf, out_hbm.at[pl.ds(wid * rows_per_w, rows_per_w)])

    return sc_kernel


@jax.jit
def kernel(x, emb_table, lin_w, lin_b):
    B, L = x.shape
    V, E = emb_table.shape
    x_flat = x.reshape(-1).astype(jnp.int32)
    w16 = lin_w.reshape(E).astype(jnp.float32)
    b16 = jnp.broadcast_to(lin_b.reshape(1), (E,)).astype(jnp.float32)
    out = _make_sc_kernel(B, L, V, E)(x_flat, emb_table, w16, b16)
    return out.reshape(B, 1)


# speedup vs baseline: 9.7074x; 1.1748x over previous
import functools

import jax
import jax.numpy as jnp
from jax import lax
from jax.experimental import pallas as pl
from jax.experimental.pallas import tpu as pltpu
from jax.experimental.pallas import tpu_sc as plsc

NC = 2
NS = 16
NW = NC * NS
LANES = 16

CHUNK_ROWS = 64
PROJ_BLOCK = 25000


def _proj_kernel_body(embv_ref, wf_ref, p_ref):
    p_ref[...] = jax.lax.dot_general(
        embv_ref[...], wf_ref[...],
        (((1,), (0,)), ((), ())),
        preferred_element_type=jnp.float32)


def _make_proj(V, E):
    rows = V * E // 128
    assert rows % PROJ_BLOCK == 0
    grid = rows // PROJ_BLOCK
    return pl.pallas_call(
        _proj_kernel_body,
        grid=(grid,),
        in_specs=[
            pl.BlockSpec((PROJ_BLOCK, 128), lambda i: (i, 0)),
            pl.BlockSpec((128, 8), lambda i: (0, 0)),
        ],
        out_specs=pl.BlockSpec((PROJ_BLOCK, 8), lambda i: (i, 0)),
        out_shape=jax.ShapeDtypeStruct((rows, 8), jnp.float32),
    )


def _make_sc_kernel(B, L, V, E):
    assert E == LANES
    rows_per_w = B // NW
    nchunks = rows_per_w // CHUNK_ROWS
    ci = CHUNK_ROWS * L
    idx_per_w = rows_per_w * L
    assert L % 8 == 0

    mesh = plsc.VectorSubcoreMesh(core_axis_name="c", subcore_axis_name="s")

    @functools.partial(
        pl.kernel,
        mesh=mesh,
        out_type=jax.ShapeDtypeStruct((B,), jnp.float32),
        compiler_params=pltpu.CompilerParams(
            needs_layout_passes=False, use_tc_tiling_on_sc=False),
        scratch_types=[
            pltpu.VMEM_SHARED((V,), jnp.float32),
            pltpu.VMEM((ci,), jnp.int32),
            pltpu.VMEM((ci,), jnp.int32),
            pltpu.VMEM((ci,), jnp.float32),
            pltpu.VMEM((ci,), jnp.float32),
            pltpu.VMEM((rows_per_w,), jnp.float32),
            pltpu.VMEM((LANES,), jnp.float32),
            pltpu.SemaphoreType.DMA,
            pltpu.SemaphoreType.DMA,
        ],
    )
    def sc_kernel(x_hbm, p_hbm, b_hbm, out_hbm,
                  p_sh, idx0, idx1, vals0, vals1, outbuf, bv, sem0, sem1):
        cid = lax.axis_index("c")
        sid = lax.axis_index("s")
        wid = sid * NC + cid
        pltpu.sync_copy(b_hbm, bv)

        @pl.when(sid == 0)
        def _():
            pltpu.sync_copy(p_hbm, p_sh)
        plsc.subcore_barrier()

        base_i = wid * idx_per_w
        iota = lax.broadcasted_iota(jnp.int32, (LANES,), 0)
        bvec = bv[...]
        idx_bufs = (idx0, idx1)
        val_bufs = (vals0, vals1)
        sems = (sem0, sem1)

        def start(c, k):
            off = base_i + c * ci
            pltpu.sync_copy(x_hbm.at[pl.ds(off, ci)], idx_bufs[k])
            return pltpu.async_copy(p_sh.at[idx_bufs[k]], val_bufs[k], sems[k])

        def reduce_chunk(c, k):
            vals_v = val_bufs[k]
            for g in range(CHUNK_ROWS // LANES):
                rowbase = (iota + g * LANES) * L

                def lbody(t, a):
                    l = t * 8
                    for u in range(8):
                        a = a + plsc.load_gather(vals_v, [rowbase + (l + u)])
                    return a

                acc = lax.fori_loop(0, L // 8, lbody,
                                    jnp.zeros((LANES,), jnp.float32))
                z = acc + bvec
                e = jnp.exp(-jnp.abs(z))
                s = jnp.where(z >= 0, 1.0 / (1.0 + e), e / (1.0 + e))
                outbuf[pl.ds(c * CHUNK_ROWS + g * LANES, LANES)] = s

        cp = start(0, 0)
        for c in range(nchunks):
            cp.wait()
            if c + 1 < nchunks:
                cp = start(c + 1, (c + 1) % 2)
            reduce_chunk(c, c % 2)

        pltpu.sync_copy(outbuf, out_hbm.at[pl.ds(wid * rows_per_w, rows_per_w)])

    return sc_kernel


@jax.jit
def kernel(x, emb_table, lin_w, lin_b):
    B, L = x.shape
    V, E = emb_table.shape
    x_flat = x.reshape(-1).astype(jnp.int32)

    k = jnp.arange(128)
    wf = jnp.where(
        (k[:, None] // E) == jnp.arange(128 // E)[None, :],
        lin_w.reshape(E)[k % E, None], 0.0,
    ).astype(jnp.float32) * (1.0 / L)

    embv = emb_table.reshape(V * E // 128, 128)
    p = _make_proj(V, E)(embv, wf).reshape(V)

    b16 = jnp.broadcast_to(lin_b.reshape(1), (LANES,)).astype(jnp.float32)
    out = _make_sc_kernel(B, L, V, E)(x_flat, p, b16)
    return out.reshape(B, 1)
